# Initial kernel scaffold; baseline (speedup 1.0000x reference)
import jax
import jax.numpy as jnp
from jax.experimental import pallas as pl


def kernel(z_e, weight):
    raise NotImplementedError("write your pallas kernel here")


# speedup vs baseline: 1.0624x; 1.0624x over previous
import functools

import jax
import jax.numpy as jnp
from jax import lax
from jax.experimental import pallas as pl
from jax.experimental.pallas import tpu as pltpu
from jax.experimental.pallas import tpu_sc as plsc

_K = 8192
_C = 256
_N = 8192
_N_BLK = 2048
_K_BLK = 512
_N_TILES = _N // _N_BLK
_K_TILES = _K // _K_BLK

_NC = 2
_NS = 16
_NW = _NC * _NS
_TOK_W = _N // _NW
_BIN_W = _K // _NS


_CHUNK_STARTS = (4096,)


def _bf16_round(x):
    return x.astype(jnp.bfloat16).astype(jnp.float32)


def _dist_body(flat_ref, w_ref, f2_ref, w2_ref, out_ref,
               acc_v, acc_i, cur_v, cur_i):
    k = pl.program_id(1)
    kb = k * _K_BLK
    fb = flat_ref[...]
    wb = w_ref[...]
    fw = lax.dot_general(fb, wb, (((1,), (1,)), ((), ())),
                         preferred_element_type=jnp.float32)
    f2 = f2_ref[...]
    w2 = w2_ref[...].reshape(1, _K_BLK)
    dist = (f2 - 2.0 * fw) + w2
    iota = lax.broadcasted_iota(jnp.int32, (_N_BLK, _K_BLK), 1)

    @pl.when(k == 0)
    def _():
        acc_v[...] = jnp.full((_N_BLK, 1), jnp.inf, jnp.float32)
        acc_i[...] = jnp.zeros((_N_BLK, 1), jnp.int32)
        cur_v[...] = jnp.full((_N_BLK, 1), jnp.inf, jnp.float32)
        cur_i[...] = jnp.zeros((_N_BLK, 1), jnp.int32)

    def update_cur(d):
        val = jnp.min(d, axis=1, keepdims=True)
        lidx = jnp.min(jnp.where(d == val, iota, _K_BLK), axis=1,
                       keepdims=True)
        win = val < cur_v[...]
        cur_v[...] = jnp.where(win, val, cur_v[...])
        cur_i[...] = jnp.where(win, kb + lidx, cur_i[...])

    def flush():
        avb = acc_v[...]
        cv = cur_v[...]
        ci = cur_i[...]
        win = (cv < avb) | ((cv == avb) & (ci < acc_i[...]))
        acc_v[...] = _bf16_round(jnp.where(win, cv, avb))
        acc_i[...] = jnp.where(win, ci, acc_i[...])
        cur_v[...] = jnp.full((_N_BLK, 1), jnp.inf, jnp.float32)
        cur_i[...] = jnp.zeros((_N_BLK, 1), jnp.int32)

    for b in _CHUNK_STARTS:
        assert b % _K_BLK == 0

        @pl.when(k == b // _K_BLK)
        def _():
            flush()

    update_cur(dist)

    @pl.when(k == _K_TILES - 1)
    def _():
        flush()
        out_ref[...] = acc_i[...][None]


def _argmin_indices(flat_bf, weight_bf, f2, w2):
    out = pl.pallas_call(
        _dist_body,
        grid=(_N_TILES, _K_TILES),
        in_specs=[
            pl.BlockSpec((_N_BLK, _C), lambda n, k: (n, 0)),
            pl.BlockSpec((_K_BLK, _C), lambda n, k: (k, 0)),
            pl.BlockSpec((_N_BLK, 1), lambda n, k: (n, 0)),
            pl.BlockSpec((1, 1, _K_BLK), lambda n, k: (k, 0, 0)),
        ],
        out_specs=pl.BlockSpec((1, _N_BLK, 1), lambda n, k: (n, 0, 0)),
        out_shape=jax.ShapeDtypeStruct((_N_TILES, _N_BLK, 1), jnp.int32),
        scratch_shapes=[
            pltpu.VMEM((_N_BLK, 1), jnp.float32),
            pltpu.VMEM((_N_BLK, 1), jnp.int32),
            pltpu.VMEM((_N_BLK, 1), jnp.float32),
            pltpu.VMEM((_N_BLK, 1), jnp.int32),
        ],
        compiler_params=pltpu.CompilerParams(
            dimension_semantics=("arbitrary", "arbitrary")),
    )(flat_bf, weight_bf, f2, w2)
    return out.reshape(_N)


def _sc_body(idx_hbm, w_hbm, q_out, cnt_out, idx_v, rows_v, ones_v,
             zeros_v, cnt_sh, sem):
    c = lax.axis_index("c")
    s = lax.axis_index("s")
    wid = s * _NC + c
    base = wid * _TOK_W

    pltpu.sync_copy(idx_hbm.at[pl.ds(base, _TOK_W)], idx_v)
    pltpu.async_copy(w_hbm.at[idx_v], rows_v, sem).wait()
    pltpu.sync_copy(rows_v, q_out.at[pl.ds(base, _TOK_W)])

    for i in range(_BIN_W // 16):
        zeros_v[pl.ds(i * 16, 16)] = jnp.zeros((16,), jnp.float32)
    for i in range(_TOK_W // 16):
        ones_v[pl.ds(i * 16, 16)] = jnp.ones((16,), jnp.float32)
    sbase = s * _BIN_W
    pltpu.sync_copy(zeros_v, cnt_sh.at[pl.ds(sbase, _BIN_W)])
    plsc.subcore_barrier()
    pltpu.sync_copy(ones_v, cnt_sh.at[idx_v], add=True)
    plsc.subcore_barrier()
    pltpu.sync_copy(cnt_sh.at[pl.ds(sbase, _BIN_W)],
                    cnt_out.at[c, pl.ds(sbase, _BIN_W)])


def _sc_gather_hist(indices, weight):
    mesh = plsc.VectorSubcoreMesh(core_axis_name="c", subcore_axis_name="s")
    fn = pl.kernel(
        _sc_body,
        out_type=[
            jax.ShapeDtypeStruct((_N, _C), jnp.float32),
            jax.ShapeDtypeStruct((_NC, _K), jnp.float32),
        ],
        mesh=mesh,
        scratch_types=[
            pltpu.VMEM((_TOK_W,), jnp.int32),
            pltpu.VMEM((_TOK_W, _C), jnp.float32),
            pltpu.VMEM((_TOK_W,), jnp.float32),
            pltpu.VMEM((_BIN_W,), jnp.float32),
            pltpu.VMEM_SHARED((_K,), jnp.float32),
            pltpu.SemaphoreType.DMA,
        ],
    )
    return fn(indices, weight)


def _fin_body(flat_ref, q_ref, cnt_ref, st_ref, loss_ref, ppl_ref, acc):
    i = pl.program_id(0)
    f = flat_ref[...]
    q = q_ref[...]
    d = q - f
    st_ref[...] = f + d
    part = jnp.sum(d * d)

    @pl.when(i == 0)
    def _():
        acc[0] = part

    @pl.when(i > 0)
    def _():
        acc[0] = acc[0] + part

    @pl.when(i == _N_TILES - 1)
    def _():
        m = acc[0] / float(_N * _C)
        loss_ref[0, 0] = m + 0.25 * m
        p = (cnt_ref[0] + cnt_ref[1]) * (1.0 / _N)
        ent = jnp.sum(p * jnp.log(p + 1e-10))
        ppl_ref[0, 0] = jnp.exp(-ent)


def _finalize(flat, quantized, counts):
    cnt = counts.reshape(_NC, _K // 128, 128)
    st, loss, ppl = pl.pallas_call(
        _fin_body,
        grid=(_N_TILES,),
        in_specs=[
            pl.BlockSpec((_N_BLK, _C), lambda i: (i, 0)),
            pl.BlockSpec((_N_BLK, _C), lambda i: (i, 0)),
            pl.BlockSpec((_NC, _K // 128, 128), lambda i: (0, 0, 0)),
        ],
        out_specs=[
            pl.BlockSpec((_N_BLK, _C), lambda i: (i, 0)),
            pl.BlockSpec(memory_space=pltpu.SMEM),
            pl.BlockSpec(memory_space=pltpu.SMEM),
        ],
        out_shape=[
            jax.ShapeDtypeStruct((_N, _C), jnp.float32),
            jax.ShapeDtypeStruct((1, 1), jnp.float32),
            jax.ShapeDtypeStruct((1, 1), jnp.float32),
        ],
        scratch_shapes=[pltpu.SMEM((1,), jnp.float32)],
        compiler_params=pltpu.CompilerParams(
            dimension_semantics=("arbitrary",)),
    )(flat, quantized, cnt)
    return st, loss[0, 0], ppl[0, 0]


def kernel(z_e, weight):
    B, C, H, W = z_e.shape
    flat = jnp.transpose(z_e, (0, 2, 3, 1)).reshape(-1, C)
    flat_bf = flat.astype(jnp.bfloat16)
    weight_bf = weight.astype(jnp.bfloat16)
    f2 = jnp.sum(z_e ** 2, axis=1).reshape(_N, 1)
    w2 = jnp.sum(weight ** 2, axis=1).reshape(_K_TILES, 1, _K_BLK)
    indices = _argmin_indices(flat_bf, weight_bf, f2, w2)
    quantized, counts = _sc_gather_hist(indices, weight)
    st, loss, perplexity = _finalize(flat, quantized, counts)
    quantized_st = jnp.transpose(st.reshape(B, H, W, C), (0, 3, 1, 2))
    return (quantized_st, loss, perplexity)


# speedup vs baseline: 1.4625x; 1.3766x over previous
import functools

import jax
import jax.numpy as jnp
from jax import lax
from jax.experimental import pallas as pl
from jax.experimental.pallas import tpu as pltpu
from jax.experimental.pallas import tpu_sc as plsc

_K = 8192
_C = 256
_N = 8192
_N_BLK = 2048
_K_BLK = 4096
_N_TILES = _N // _N_BLK
_K_TILES = _K // _K_BLK

_NC = 2
_NS = 16
_NW = _NC * _NS
_TOK_W = _N // _NW
_BIN_W = _K // _NS


_CHUNK_STARTS = (4096,)


def _bf16_round(x):
    return x.astype(jnp.bfloat16).astype(jnp.float32)


def _dist_body(flat_ref, w_ref, f2_ref, w2_ref, out_ref,
               acc_v, acc_i, cur_v, cur_i):
    k = pl.program_id(1)
    kb = k * _K_BLK
    fb = flat_ref[...]
    wb = w_ref[...]
    fw2 = lax.dot_general(fb, wb, (((1,), (1,)), ((), ())),
                          preferred_element_type=jnp.float32)
    f2 = f2_ref[...]
    w2 = w2_ref[...].reshape(1, _K_BLK)
    dist = (f2 - fw2) + w2
    iota = lax.broadcasted_iota(
        jnp.int32, (1, _K_BLK), 1).astype(jnp.float32)

    @pl.when(k == 0)
    def _():
        acc_v[...] = jnp.full((_N_BLK, 1), jnp.inf, jnp.float32)
        acc_i[...] = jnp.zeros((_N_BLK, 1), jnp.int32)
        cur_v[...] = jnp.full((_N_BLK, 1), jnp.inf, jnp.float32)
        cur_i[...] = jnp.zeros((_N_BLK, 1), jnp.int32)

    def update_cur(d):
        val = jnp.min(d, axis=1, keepdims=True)
        lidx = jnp.min(jnp.where(d == val, iota, float(_K_BLK)), axis=1,
                       keepdims=True).astype(jnp.int32)
        win = val < cur_v[...]
        cur_v[...] = jnp.where(win, val, cur_v[...])
        cur_i[...] = jnp.where(win, kb + lidx, cur_i[...])

    def flush():
        avb = acc_v[...]
        cv = cur_v[...]
        ci = cur_i[...]
        win = (cv < avb) | ((cv == avb) & (ci < acc_i[...]))
        acc_v[...] = _bf16_round(jnp.where(win, cv, avb))
        acc_i[...] = jnp.where(win, ci, acc_i[...])
        cur_v[...] = jnp.full((_N_BLK, 1), jnp.inf, jnp.float32)
        cur_i[...] = jnp.zeros((_N_BLK, 1), jnp.int32)

    for b in _CHUNK_STARTS:
        assert b % _K_BLK == 0

        @pl.when(k == b // _K_BLK)
        def _():
            flush()

    update_cur(dist)

    @pl.when(k == _K_TILES - 1)
    def _():
        flush()
        out_ref[...] = acc_i[...][None]


def _argmin_indices(flat_bf, weight_bf, f2, w2):
    out = pl.pallas_call(
        _dist_body,
        grid=(_N_TILES, _K_TILES),
        in_specs=[
            pl.BlockSpec((_N_BLK, _C), lambda n, k: (n, 0)),
            pl.BlockSpec((_K_BLK, _C), lambda n, k: (k, 0)),
            pl.BlockSpec((_N_BLK, 1), lambda n, k: (n, 0)),
            pl.BlockSpec((1, 1, _K_BLK), lambda n, k: (k, 0, 0)),
        ],
        out_specs=pl.BlockSpec((1, _N_BLK, 1), lambda n, k: (n, 0, 0)),
        out_shape=jax.ShapeDtypeStruct((_N_TILES, _N_BLK, 1), jnp.int32),
        scratch_shapes=[
            pltpu.VMEM((_N_BLK, 1), jnp.float32),
            pltpu.VMEM((_N_BLK, 1), jnp.int32),
            pltpu.VMEM((_N_BLK, 1), jnp.float32),
            pltpu.VMEM((_N_BLK, 1), jnp.int32),
        ],
        compiler_params=pltpu.CompilerParams(
            dimension_semantics=("arbitrary", "arbitrary")),
    )(flat_bf, weight_bf, f2, w2)
    return out.reshape(_N)


def _sc_body(idx_hbm, w_hbm, q_out, cnt_out, idx_v, rows_v, ones_v,
             zeros_v, cnt_sh, sem):
    c = lax.axis_index("c")
    s = lax.axis_index("s")
    wid = s * _NC + c
    base = wid * _TOK_W

    pltpu.sync_copy(idx_hbm.at[pl.ds(base, _TOK_W)], idx_v)
    pltpu.async_copy(w_hbm.at[idx_v], rows_v, sem).wait()
    pltpu.sync_copy(rows_v, q_out.at[pl.ds(base, _TOK_W)])

    for i in range(_BIN_W // 16):
        zeros_v[pl.ds(i * 16, 16)] = jnp.zeros((16,), jnp.float32)
    for i in range(_TOK_W // 16):
        ones_v[pl.ds(i * 16, 16)] = jnp.ones((16,), jnp.float32)
    sbase = s * _BIN_W
    pltpu.sync_copy(zeros_v, cnt_sh.at[pl.ds(sbase, _BIN_W)])
    plsc.subcore_barrier()
    pltpu.sync_copy(ones_v, cnt_sh.at[idx_v], add=True)
    plsc.subcore_barrier()
    pltpu.sync_copy(cnt_sh.at[pl.ds(sbase, _BIN_W)],
                    cnt_out.at[c, pl.ds(sbase, _BIN_W)])


def _sc_gather_hist(indices, weight):
    mesh = plsc.VectorSubcoreMesh(core_axis_name="c", subcore_axis_name="s")
    fn = pl.kernel(
        _sc_body,
        out_type=[
            jax.ShapeDtypeStruct((_N, _C), jnp.float32),
            jax.ShapeDtypeStruct((_NC, _K), jnp.float32),
        ],
        mesh=mesh,
        scratch_types=[
            pltpu.VMEM((_TOK_W,), jnp.int32),
            pltpu.VMEM((_TOK_W, _C), jnp.float32),
            pltpu.VMEM((_TOK_W,), jnp.float32),
            pltpu.VMEM((_BIN_W,), jnp.float32),
            pltpu.VMEM_SHARED((_K,), jnp.float32),
            pltpu.SemaphoreType.DMA,
        ],
    )
    return fn(indices, weight)


def _fin_body(flat_ref, q_ref, cnt_ref, st_ref, loss_ref, ppl_ref, acc):
    i = pl.program_id(0)
    f = flat_ref[...]
    q = q_ref[...]
    d = q - f
    st_ref[...] = f + d
    part = jnp.sum(d * d)

    @pl.when(i == 0)
    def _():
        acc[0] = part

    @pl.when(i > 0)
    def _():
        acc[0] = acc[0] + part

    @pl.when(i == _N_TILES - 1)
    def _():
        m = acc[0] / float(_N * _C)
        loss_ref[0, 0] = m + 0.25 * m
        p = (cnt_ref[0] + cnt_ref[1]) * (1.0 / _N)
        ent = jnp.sum(p * jnp.log(p + 1e-10))
        ppl_ref[0, 0] = jnp.exp(-ent)


def _finalize(flat, quantized, counts):
    cnt = counts.reshape(_NC, _K // 128, 128)
    st, loss, ppl = pl.pallas_call(
        _fin_body,
        grid=(_N_TILES,),
        in_specs=[
            pl.BlockSpec((_N_BLK, _C), lambda i: (i, 0)),
            pl.BlockSpec((_N_BLK, _C), lambda i: (i, 0)),
            pl.BlockSpec((_NC, _K // 128, 128), lambda i: (0, 0, 0)),
        ],
        out_specs=[
            pl.BlockSpec((_N_BLK, _C), lambda i: (i, 0)),
            pl.BlockSpec(memory_space=pltpu.SMEM),
            pl.BlockSpec(memory_space=pltpu.SMEM),
        ],
        out_shape=[
            jax.ShapeDtypeStruct((_N, _C), jnp.float32),
            jax.ShapeDtypeStruct((1, 1), jnp.float32),
            jax.ShapeDtypeStruct((1, 1), jnp.float32),
        ],
        scratch_shapes=[pltpu.SMEM((1,), jnp.float32)],
        compiler_params=pltpu.CompilerParams(
            dimension_semantics=("arbitrary",)),
    )(flat, quantized, cnt)
    return st, loss[0, 0], ppl[0, 0]


def kernel(z_e, weight):
    B, C, H, W = z_e.shape
    flat = jnp.transpose(z_e, (0, 2, 3, 1)).reshape(-1, C)
    flat_bf = (2.0 * flat).astype(jnp.bfloat16)
    weight_bf = weight.astype(jnp.bfloat16)
    f2 = jnp.sum(z_e ** 2, axis=1).reshape(_N, 1)
    w2 = jnp.sum(weight ** 2, axis=1).reshape(_K_TILES, 1, _K_BLK)
    indices = _argmin_indices(flat_bf, weight_bf, f2, w2)
    quantized, counts = _sc_gather_hist(indices, weight)
    st, loss, perplexity = _finalize(flat, quantized, counts)
    quantized_st = jnp.transpose(st.reshape(B, H, W, C), (0, 3, 1, 2))
    return (quantized_st, loss, perplexity)


# speedup vs baseline: 1.5326x; 1.0480x over previous
import functools

import jax
import jax.numpy as jnp
from jax import lax
from jax.experimental import pallas as pl
from jax.experimental.pallas import tpu as pltpu
from jax.experimental.pallas import tpu_sc as plsc

_K = 8192
_C = 256
_N = 8192
_N_BLK = 2048
_K_BLK = 4096
_N_TILES = _N // _N_BLK
_K_TILES = _K // _K_BLK

_NC = 2
_NS = 16
_NW = _NC * _NS
_TOK_W = _N // _NW
_BIN_W = _K // _NS


_CHUNK_STARTS = (4096,)


def _bf16_round(x):
    return x.astype(jnp.bfloat16).astype(jnp.float32)


def _dist_body(flat_ref, w_ref, f2_ref, w2_ref, out_ref,
               acc_v, acc_i, cur_v, cur_i):
    k = pl.program_id(1)
    kb = k * _K_BLK
    fb = (2.0 * flat_ref[...]).astype(jnp.bfloat16)
    wb = w_ref[...].astype(jnp.bfloat16)
    fw2 = lax.dot_general(fb, wb, (((1,), (1,)), ((), ())),
                          preferred_element_type=jnp.float32)
    f2 = f2_ref[...]
    w2 = w2_ref[...].reshape(1, _K_BLK)
    dist = (f2 - fw2) + w2
    iota = lax.broadcasted_iota(
        jnp.int32, (1, _K_BLK), 1).astype(jnp.float32)

    @pl.when(k == 0)
    def _():
        acc_v[...] = jnp.full((_N_BLK, 1), jnp.inf, jnp.float32)
        acc_i[...] = jnp.zeros((_N_BLK, 1), jnp.int32)
        cur_v[...] = jnp.full((_N_BLK, 1), jnp.inf, jnp.float32)
        cur_i[...] = jnp.zeros((_N_BLK, 1), jnp.int32)

    def update_cur(d):
        val = jnp.min(d, axis=1, keepdims=True)
        lidx = jnp.min(jnp.where(d == val, iota, float(_K_BLK)), axis=1,
                       keepdims=True).astype(jnp.int32)
        win = val < cur_v[...]
        cur_v[...] = jnp.where(win, val, cur_v[...])
        cur_i[...] = jnp.where(win, kb + lidx, cur_i[...])

    def flush():
        avb = acc_v[...]
        cv = cur_v[...]
        ci = cur_i[...]
        win = (cv < avb) | ((cv == avb) & (ci < acc_i[...]))
        acc_v[...] = _bf16_round(jnp.where(win, cv, avb))
        acc_i[...] = jnp.where(win, ci, acc_i[...])
        cur_v[...] = jnp.full((_N_BLK, 1), jnp.inf, jnp.float32)
        cur_i[...] = jnp.zeros((_N_BLK, 1), jnp.int32)

    for b in _CHUNK_STARTS:
        assert b % _K_BLK == 0

        @pl.when(k == b // _K_BLK)
        def _():
            flush()

    update_cur(dist)

    @pl.when(k == _K_TILES - 1)
    def _():
        flush()
        out_ref[...] = acc_i[...][None]


def _argmin_indices(flat_bf, weight_bf, f2, w2):
    out = pl.pallas_call(
        _dist_body,
        grid=(_N_TILES, _K_TILES),
        in_specs=[
            pl.BlockSpec((_N_BLK, _C), lambda n, k: (n, 0)),
            pl.BlockSpec((_K_BLK, _C), lambda n, k: (k, 0)),
            pl.BlockSpec((_N_BLK, 1), lambda n, k: (n, 0)),
            pl.BlockSpec((1, 1, _K_BLK), lambda n, k: (k, 0, 0)),
        ],
        out_specs=pl.BlockSpec((1, _N_BLK, 1), lambda n, k: (n, 0, 0)),
        out_shape=jax.ShapeDtypeStruct((_N_TILES, _N_BLK, 1), jnp.int32),
        scratch_shapes=[
            pltpu.VMEM((_N_BLK, 1), jnp.float32),
            pltpu.VMEM((_N_BLK, 1), jnp.int32),
            pltpu.VMEM((_N_BLK, 1), jnp.float32),
            pltpu.VMEM((_N_BLK, 1), jnp.int32),
        ],
        compiler_params=pltpu.CompilerParams(
            dimension_semantics=("arbitrary", "arbitrary")),
    )(flat_bf, weight_bf, f2, w2)
    return out.reshape(_N)


def _sc_body(idx_hbm, w_hbm, q_out, cnt_out, idx_v, rows_v, ones_v,
             zeros_v, cnt_sh, sem):
    c = lax.axis_index("c")
    s = lax.axis_index("s")
    wid = s * _NC + c
    base = wid * _TOK_W

    pltpu.sync_copy(idx_hbm.at[pl.ds(base, _TOK_W)], idx_v)
    pltpu.async_copy(w_hbm.at[idx_v], rows_v, sem).wait()
    pltpu.sync_copy(rows_v, q_out.at[pl.ds(base, _TOK_W)])

    for i in range(_BIN_W // 16):
        zeros_v[pl.ds(i * 16, 16)] = jnp.zeros((16,), jnp.float32)
    for i in range(_TOK_W // 16):
        ones_v[pl.ds(i * 16, 16)] = jnp.ones((16,), jnp.float32)
    sbase = s * _BIN_W
    pltpu.sync_copy(zeros_v, cnt_sh.at[pl.ds(sbase, _BIN_W)])
    plsc.subcore_barrier()
    pltpu.sync_copy(ones_v, cnt_sh.at[idx_v], add=True)
    plsc.subcore_barrier()
    pltpu.sync_copy(cnt_sh.at[pl.ds(sbase, _BIN_W)],
                    cnt_out.at[c, pl.ds(sbase, _BIN_W)])


def _sc_gather_hist(indices, weight):
    mesh = plsc.VectorSubcoreMesh(core_axis_name="c", subcore_axis_name="s")
    fn = pl.kernel(
        _sc_body,
        out_type=[
            jax.ShapeDtypeStruct((_N, _C), jnp.float32),
            jax.ShapeDtypeStruct((_NC, _K), jnp.float32),
        ],
        mesh=mesh,
        scratch_types=[
            pltpu.VMEM((_TOK_W,), jnp.int32),
            pltpu.VMEM((_TOK_W, _C), jnp.float32),
            pltpu.VMEM((_TOK_W,), jnp.float32),
            pltpu.VMEM((_BIN_W,), jnp.float32),
            pltpu.VMEM_SHARED((_K,), jnp.float32),
            pltpu.SemaphoreType.DMA,
        ],
    )
    return fn(indices, weight)


def _fin_body(flat_ref, q_ref, cnt_ref, st_ref, loss_ref, ppl_ref, acc):
    i = pl.program_id(0)
    f = flat_ref[...]
    q = q_ref[...]
    d = q - f
    st_ref[...] = f + d
    part = jnp.sum(d * d)

    @pl.when(i == 0)
    def _():
        acc[0] = part

    @pl.when(i > 0)
    def _():
        acc[0] = acc[0] + part

    @pl.when(i == _N_TILES - 1)
    def _():
        m = acc[0] / float(_N * _C)
        loss_ref[0, 0] = m + 0.25 * m
        p = (cnt_ref[0] + cnt_ref[1]) * (1.0 / _N)
        ent = jnp.sum(p * jnp.log(p + 1e-10))
        ppl_ref[0, 0] = jnp.exp(-ent)


def _finalize(flat, quantized, counts):
    cnt = counts.reshape(_NC, _K // 128, 128)
    st, loss, ppl = pl.pallas_call(
        _fin_body,
        grid=(_N_TILES,),
        in_specs=[
            pl.BlockSpec((_N_BLK, _C), lambda i: (i, 0)),
            pl.BlockSpec((_N_BLK, _C), lambda i: (i, 0)),
            pl.BlockSpec((_NC, _K // 128, 128), lambda i: (0, 0, 0)),
        ],
        out_specs=[
            pl.BlockSpec((_N_BLK, _C), lambda i: (i, 0)),
            pl.BlockSpec(memory_space=pltpu.SMEM),
            pl.BlockSpec(memory_space=pltpu.SMEM),
        ],
        out_shape=[
            jax.ShapeDtypeStruct((_N, _C), jnp.float32),
            jax.ShapeDtypeStruct((1, 1), jnp.float32),
            jax.ShapeDtypeStruct((1, 1), jnp.float32),
        ],
        scratch_shapes=[pltpu.SMEM((1,), jnp.float32)],
        compiler_params=pltpu.CompilerParams(
            dimension_semantics=("arbitrary",)),
    )(flat, quantized, cnt)
    return st, loss[0, 0], ppl[0, 0]


def kernel(z_e, weight):
    B, C, H, W = z_e.shape
    flat = jnp.transpose(z_e, (0, 2, 3, 1)).reshape(-1, C)
    f2 = jnp.sum(z_e ** 2, axis=1).reshape(_N, 1)
    w2 = jnp.sum(weight ** 2, axis=1).reshape(_K_TILES, 1, _K_BLK)
    indices = _argmin_indices(flat, weight, f2, w2)
    quantized, counts = _sc_gather_hist(indices, weight)
    st, loss, perplexity = _finalize(flat, quantized, counts)
    quantized_st = jnp.transpose(st.reshape(B, H, W, C), (0, 3, 1, 2))
    return (quantized_st, loss, perplexity)
